# baseline (speedup 1.0000x reference)
import functools

import jax
import jax.numpy as jnp
from jax import lax
from jax.experimental import pallas as pl
from jax.experimental.pallas import tpu as pltpu
from jax.experimental.pallas import tpu_sc as plsc

BATCH = 16384
LANES = 16
NUM_CORES = 2
NUM_SUBCORES = 16
NW = NUM_CORES * NUM_SUBCORES
B_PER_W = BATCH // NW
HALF = B_PER_W // 2
CHUNK = 128
N_CHUNK = B_PER_W // CHUNK


@functools.partial(
    pl.kernel,
    mesh=plsc.VectorSubcoreMesh(core_axis_name="c", subcore_axis_name="s"),
    out_type=jax.ShapeDtypeStruct((BATCH,), jnp.float32),
    scratch_types=[
        pltpu.VMEM((B_PER_W,), jnp.int32),
        pltpu.VMEM((B_PER_W,), jnp.float32),
        pltpu.SemaphoreType.DMA((2,)),
        pltpu.SemaphoreType.DMA((N_CHUNK,)),
        pltpu.SemaphoreType.DMA,
    ],
)
def _emb_sigmoid(w_hbm, idx_hbm, out_hbm, idx_v, val_v, sem_i, sem_g, sem_o):
    wid = lax.axis_index("s") * NUM_CORES + lax.axis_index("c")
    base = wid * B_PER_W
    idx_copies = [
        pltpu.async_copy(
            idx_hbm.at[pl.ds(base + h * HALF, HALF)],
            idx_v.at[pl.ds(h * HALF, HALF)],
            sem_i.at[h],
        )
        for h in range(2)
    ]
    gathers = []
    for h in range(2):
        idx_copies[h].wait()
        for j in (2 * h, 2 * h + 1):
            gathers.append(
                pltpu.async_copy(
                    w_hbm.at[idx_v.at[pl.ds(j * CHUNK, CHUNK)]],
                    val_v.at[pl.ds(j * CHUNK, CHUNK)],
                    sem_g.at[j],
                )
            )
    stores = []
    for j in range(N_CHUNK):
        gathers[j].wait()
        for i in range(CHUNK // LANES):
            o = j * CHUNK + i * LANES
            x = val_v[pl.ds(o, LANES)]
            val_v[pl.ds(o, LANES)] = 1.0 / (1.0 + jnp.exp(-x))
        stores.append(
            pltpu.async_copy(
                val_v.at[pl.ds(j * CHUNK, CHUNK)],
                out_hbm.at[pl.ds(base + j * CHUNK, CHUNK)],
                sem_o,
            )
        )
    for c in stores:
        c.wait()


def kernel(idx, W):
    out = _emb_sigmoid(W.reshape(-1), idx.astype(jnp.int32))
    return out.reshape(BATCH, 1)


# speedup vs baseline: 1.0011x; 1.0011x over previous
import functools

import jax
import jax.numpy as jnp
from jax import lax
from jax.experimental import pallas as pl
from jax.experimental.pallas import tpu as pltpu
from jax.experimental.pallas import tpu_sc as plsc

BATCH = 16384
LANES = 16
NUM_CORES = 2
NUM_SUBCORES = 16
NW = NUM_CORES * NUM_SUBCORES
B_PER_W = BATCH // NW
HALF = B_PER_W // 2


@functools.partial(
    pl.kernel,
    mesh=plsc.VectorSubcoreMesh(core_axis_name="c", subcore_axis_name="s"),
    out_type=jax.ShapeDtypeStruct((BATCH,), jnp.float32),
    scratch_types=[
        pltpu.VMEM((B_PER_W,), jnp.int32),
        pltpu.VMEM((B_PER_W,), jnp.float32),
        pltpu.SemaphoreType.DMA((2,)),
        pltpu.SemaphoreType.DMA((2,)),
        pltpu.SemaphoreType.DMA,
    ],
)
def _emb_sigmoid(w_hbm, idx_hbm, out_hbm, idx_v, val_v, sem_i, sem_g, sem_o):
    wid = lax.axis_index("s") * NUM_CORES + lax.axis_index("c")
    base = wid * B_PER_W
    idx_copies = [
        pltpu.async_copy(
            idx_hbm.at[pl.ds(base + h * HALF, HALF)],
            idx_v.at[pl.ds(h * HALF, HALF)],
            sem_i.at[h],
        )
        for h in range(2)
    ]
    gathers = []
    for h in range(2):
        idx_copies[h].wait()
        gathers.append(
            pltpu.async_copy(
                w_hbm.at[idx_v.at[pl.ds(h * HALF, HALF)]],
                val_v.at[pl.ds(h * HALF, HALF)],
                sem_g.at[h],
            )
        )
    stores = []
    for h in range(2):
        gathers[h].wait()
        for i in range(HALF // LANES):
            o = h * HALF + i * LANES
            x = val_v[pl.ds(o, LANES)]
            val_v[pl.ds(o, LANES)] = 1.0 / (1.0 + jnp.exp(-x))
        stores.append(
            pltpu.async_copy(
                val_v.at[pl.ds(h * HALF, HALF)],
                out_hbm.at[pl.ds(base + h * HALF, HALF)],
                sem_o,
            )
        )
    for c in stores:
        c.wait()


def kernel(idx, W):
    out = _emb_sigmoid(W.reshape(-1), idx.astype(jnp.int32))
    return out.reshape(BATCH, 1)
